# Initial kernel scaffold; baseline (speedup 1.0000x reference)
import functools

import jax
import jax.numpy as jnp
from jax import lax
from jax.experimental import pallas as pl
from jax.experimental.pallas import tpu as pltpu
from jax.experimental.pallas import tpu_sc as plsc

EMBEDDING_DIM = 32


def _build_sc_gather(B, D, num_cores, num_subcores, chunk):
    nw = num_cores * num_subcores
    b_per_w = B // nw
    n_chunks = b_per_w // chunk
    mesh = plsc.VectorSubcoreMesh(core_axis_name="c", subcore_axis_name="s")

    @functools.partial(
        pl.kernel,
        mesh=mesh,
        out_type=jax.ShapeDtypeStruct((B, D), jnp.float32),
        scratch_types=[
            pltpu.VMEM((chunk,), jnp.int32),
            pltpu.VMEM((chunk, D), jnp.float32),
            pltpu.SemaphoreType.DMA,
        ],
    )
    def emb(idx_hbm, table_hbm, out_hbm, idx_v, rows_v, gsem):
        wid = lax.axis_index("s") * num_cores + lax.axis_index("c")
        base = wid * b_per_w

        def body(i, carry):
            start = base + i * chunk
            pltpu.sync_copy(idx_hbm.at[pl.ds(start, chunk)], idx_v)
            pltpu.async_copy(table_hbm.at[idx_v], rows_v, gsem).wait()
            pltpu.sync_copy(rows_v, out_hbm.at[pl.ds(start, chunk)])
            return carry

        lax.fori_loop(0, n_chunks, body, 0)

    return emb


def kernel(x, weight):
    B = x.shape[0] * x.shape[1]
    D = weight.shape[1]
    idx = x.reshape(B).astype(jnp.int32)
    emb = _build_sc_gather(B, D, num_cores=2, num_subcores=16, chunk=3200)
    out = emb(idx, weight)
    return out.reshape(x.shape[0], x.shape[1], D)


# speedup vs baseline: 1.1113x; 1.1113x over previous
import functools

import jax
import jax.numpy as jnp
from jax import lax
from jax.experimental import pallas as pl
from jax.experimental.pallas import tpu as pltpu
from jax.experimental.pallas import tpu_sc as plsc

EMBEDDING_DIM = 32


def _build_sc_gather(B, D, num_cores, num_subcores, chunk):
    nw = num_cores * num_subcores
    b_per_w = B // nw
    n_chunks = b_per_w // chunk
    mesh = plsc.VectorSubcoreMesh(core_axis_name="c", subcore_axis_name="s")

    @functools.partial(
        pl.kernel,
        mesh=mesh,
        out_type=jax.ShapeDtypeStruct((B, D), jnp.float32),
        scratch_types=[
            pltpu.VMEM((chunk,), jnp.int32),
            pltpu.VMEM((chunk, D), jnp.float32),
            pltpu.SemaphoreType.DMA,
        ],
        compiler_params=pltpu.CompilerParams(use_tc_tiling_on_sc=False),
    )
    def emb(idx_hbm, table_hbm, out_hbm, idx_v, rows_v, gsem):
        wid = lax.axis_index("s") * num_cores + lax.axis_index("c")
        base = wid * b_per_w

        def body(i, carry):
            start = base + i * chunk
            pltpu.sync_copy(idx_hbm.at[pl.ds(start, chunk)], idx_v)
            pltpu.async_copy(table_hbm.at[idx_v], rows_v, gsem).wait()
            pltpu.sync_copy(rows_v, out_hbm.at[pl.ds(start, chunk)])
            return carry

        lax.fori_loop(0, n_chunks, body, 0)

    return emb


def kernel(x, weight):
    B = x.shape[0] * x.shape[1]
    D = weight.shape[1]
    idx = x.reshape(B).astype(jnp.int32)
    emb = _build_sc_gather(B, D, num_cores=2, num_subcores=16, chunk=3200)
    out = emb(idx, weight)
    return out.reshape(x.shape[0], x.shape[1], D)
